# baseline (speedup 1.0000x reference)
import functools

import jax
import jax.numpy as jnp
from jax.experimental import pallas as pl
from jax.experimental.pallas import tpu as pltpu

_EPS = 1e-16
_SEL0 = 64
_SEL1 = 128
_RB = 256


def _dist_kernel(xi_ref, xt_ref, tf_ref, box_ref, m0_ref, m1_ref):
    rb = pl.program_id(1)
    n = xt_ref.shape[2]
    box = box_ref[0, 0]
    inv_box = 1.0 / box
    xi = xi_ref[0]
    acc = jnp.zeros((_RB, n), jnp.float32)
    for k in range(3):
        a = xi[:, k : k + 1]
        bj = xt_ref[0, k : k + 1, :]
        t = a - bj + _EPS
        t = t - box * jnp.round(t * inv_box)
        acc = acc + t * t
    row_i = jax.lax.broadcasted_iota(jnp.int32, (_RB, n), 0) + rb * _RB
    col_j = jax.lax.broadcasted_iota(jnp.int32, (_RB, n), 1)
    diag = row_i == col_j
    is0 = tf_ref[0] == 0.0
    inf = jnp.float32(jnp.inf)
    m0_ref[0] = jnp.where(diag | (~is0), inf, acc)
    m1_ref[0] = jnp.where(diag | is0, inf, acc)


def _mlp_kernel(desc_ref, tf_ref,
                w00, b00, w10, b10, w20, b20,
                w01, b01, w11, b11, w21, b21,
                out_ref):
    x = desc_ref[0]

    def mlp(x, w0, b0, w1, b1, w2, b2):
        h = jnp.tanh(jnp.dot(x, w0[...], preferred_element_type=jnp.float32) + b0[...])
        h = jnp.tanh(jnp.dot(h, w1[...], preferred_element_type=jnp.float32) + b1[...])
        return jnp.dot(h, w2[...], preferred_element_type=jnp.float32) + b2[...]

    e0 = mlp(x, w00, b00, w10, b10, w20, b20)
    e1 = mlp(x, w01, b01, w11, b11, w21, b21)
    t = tf_ref[0]
    e = jnp.where(t == 0.0, e0, e1)
    out_ref[0, 0, :] = jnp.broadcast_to(jnp.sum(e), (128,))


def kernel(xyz, box_size, W0_t0, b0_t0, W1_t0, b1_t0, W2_t0, b2_t0,
           W0_t1, b0_t1, W1_t1, b1_t1, W2_t1, b2_t1, atomtypes):
    B, N, _ = xyz.shape
    D = _SEL0 + _SEL1
    xt = jnp.transpose(xyz, (0, 2, 1))
    tf = atomtypes.astype(jnp.float32).reshape(B, 1, N)
    box2 = box_size.reshape(1, 1)

    m0, m1 = pl.pallas_call(
        _dist_kernel,
        grid=(B, N // _RB),
        in_specs=[
            pl.BlockSpec((1, _RB, 3), lambda b, r: (b, r, 0)),
            pl.BlockSpec((1, 3, N), lambda b, r: (b, 0, 0)),
            pl.BlockSpec((1, 1, N), lambda b, r: (b, 0, 0)),
            pl.BlockSpec((1, 1), lambda b, r: (0, 0)),
        ],
        out_specs=[
            pl.BlockSpec((1, _RB, N), lambda b, r: (b, r, 0)),
            pl.BlockSpec((1, _RB, N), lambda b, r: (b, r, 0)),
        ],
        out_shape=[
            jax.ShapeDtypeStruct((B, N, N), jnp.float32),
            jax.ShapeDtypeStruct((B, N, N), jnp.float32),
        ],
    )(xyz, xt, tf, box2)

    s0 = jnp.sort(m0, axis=2)[:, :, :_SEL0]
    s1 = jnp.sort(m1, axis=2)[:, :, :_SEL1]
    sq = jnp.concatenate([s0, s1], axis=2)
    desc = 1.0 / (jnp.sqrt(sq) + _EPS)

    tcol = atomtypes.astype(jnp.float32).reshape(B, N, 1)

    out = pl.pallas_call(
        _mlp_kernel,
        grid=(B,),
        in_specs=[
            pl.BlockSpec((1, N, D), lambda b: (b, 0, 0)),
            pl.BlockSpec((1, N, 1), lambda b: (b, 0, 0)),
        ] + [pl.BlockSpec(w.shape, functools.partial(lambda n, b: (0,) * n, len(w.shape)))
             for w in (W0_t0, b0_t0, W1_t0, b1_t0, W2_t0, b2_t0,
                       W0_t1, b0_t1, W1_t1, b1_t1, W2_t1, b2_t1)],
        out_specs=pl.BlockSpec((1, 1, 128), lambda b: (b, 0, 0)),
        out_shape=jax.ShapeDtypeStruct((B, 1, 128), jnp.float32),
    )(desc, tcol, W0_t0, b0_t0, W1_t0, b1_t0, W2_t0, b2_t0,
      W0_t1, b0_t1, W1_t1, b1_t1, W2_t1, b2_t1)

    return out[:, 0, 0]
